# baseline (speedup 1.0000x reference)
import functools

import jax
import jax.numpy as jnp
from jax import lax
from jax.experimental import pallas as pl
from jax.experimental.pallas import tpu as pltpu
from jax.experimental.pallas import tpu_sc as plsc

NUM_CORES = 2
NUM_SUBCORES = 16
NUM_WORKERS = NUM_CORES * NUM_SUBCORES

BATCH = 16384
HIST = 50
EMBED_DIM = 64
BLK = 128
NUM_BLOCKS = BATCH // BLK
BLOCKS_PER_WORKER = NUM_BLOCKS // NUM_WORKERS

_mesh = plsc.VectorSubcoreMesh(core_axis_name="c", subcore_axis_name="s")


@functools.partial(
    pl.kernel,
    out_type=jax.ShapeDtypeStruct((HIST, EMBED_DIM, BATCH), jnp.float32),
    mesh=_mesh,
    scratch_types=[
        pltpu.VMEM((BLK * HIST,), jnp.int32),
        pltpu.VMEM((2, BLK), jnp.int32),
        pltpu.VMEM((2, BLK, BLK), jnp.float32),
        pltpu.VMEM((2, EMBED_DIM, BLK), jnp.float32),
        pltpu.SemaphoreType.DMA,
        pltpu.SemaphoreType.DMA,
        pltpu.SemaphoreType.DMA,
        pltpu.SemaphoreType.DMA,
    ],
    compiler_params=pltpu.CompilerParams(
        use_tc_tiling_on_sc=True, needs_layout_passes=False
    ),
)
def _sc_gather(idx_hbm, tab_hbm, out_hbm, idx_blk, ipair, prows, otile,
               gsem0, gsem1, wsem0, wsem1):
    wid = lax.axis_index("s") * NUM_CORES + lax.axis_index("c")
    gsems = (gsem0, gsem1)
    wsems = (wsem0, wsem1)
    lane = lax.iota(jnp.int32, 16)

    def build_indices(h, s):
        @pl.loop(0, BLK // 16)
        def _g(g):
            j0 = g * 16
            raw = plsc.load_gather(idx_blk, [(j0 + lane) * HIST + h])
            ipair[s, pl.ds(j0, 16)] = raw

    def start_gather(s):
        pltpu.async_copy(tab_hbm.at[ipair.at[s]], prows.at[s], gsems[s])

    def wait_gather(s):
        pltpu.make_async_copy(
            tab_hbm.at[ipair.at[s]], prows.at[s], gsems[s]
        ).wait()

    def transpose_select(s):
        for g in range(BLK // 16):
            j0 = g * 16
            rowv = j0 + lane
            for d in range(EMBED_DIM):
                vals = plsc.load_gather(prows.at[s], [rowv, lane * 0 + d])
                otile[s, d, pl.ds(j0, 16)] = vals

    def start_write(h, b0, s):
        pltpu.async_copy(
            otile.at[s], out_hbm.at[h, :, pl.ds(b0, BLK)], wsems[s]
        )

    def wait_write(h, b0, s):
        pltpu.make_async_copy(
            otile.at[s], out_hbm.at[h, :, pl.ds(b0, BLK)], wsems[s]
        ).wait()

    @pl.loop(0, BLOCKS_PER_WORKER)
    def _blk(bi):
        blk_id = wid * BLOCKS_PER_WORKER + bi
        b0 = blk_id * BLK
        pltpu.sync_copy(idx_hbm.at[pl.ds(b0 * HIST, BLK * HIST)], idx_blk)

        build_indices(0, 0)
        start_gather(0)

        @pl.loop(0, HIST, step=2)
        def _h(h0):
            build_indices(h0 + 1, 1)
            start_gather(1)
            wait_gather(0)

            @pl.when(h0 >= 2)
            def _():
                wait_write(h0 - 2, b0, 0)

            transpose_select(0)
            start_write(h0, b0, 0)

            @pl.when(h0 + 2 < HIST)
            def _():
                build_indices(h0 + 2, 0)
                start_gather(0)

            wait_gather(1)

            @pl.when(h0 >= 1)
            def _():
                wait_write(h0 - 1, b0, 1)

            transpose_select(1)
            start_write(h0 + 1, b0, 1)

        wait_write(HIST - 2, b0, 0)
        wait_write(HIST - 1, b0, 1)


def kernel(indices, table):
    idx = indices.reshape(-1).astype(jnp.int32)
    tabp = jnp.pad(table, ((0, 0), (0, 64)))
    out_t = _sc_gather(idx, tabp)
    return jnp.transpose(out_t, (2, 0, 1))


# speedup vs baseline: 1.6482x; 1.6482x over previous
import functools

import jax
import jax.numpy as jnp
from jax import lax
from jax.experimental import pallas as pl
from jax.experimental.pallas import tpu as pltpu
from jax.experimental.pallas import tpu_sc as plsc

NUM_CORES = 2
NUM_SUBCORES = 16
NUM_WORKERS = NUM_CORES * NUM_SUBCORES

BATCH = 16384
HIST = 50
EMBED_DIM = 64
BLK = 128
NUM_BLOCKS = BATCH // BLK
BLOCKS_PER_WORKER = NUM_BLOCKS // NUM_WORKERS

_mesh = plsc.VectorSubcoreMesh(core_axis_name="c", subcore_axis_name="s")


@functools.partial(
    pl.kernel,
    out_type=jax.ShapeDtypeStruct((HIST, EMBED_DIM, BATCH), jnp.float32),
    mesh=_mesh,
    scratch_types=[
        pltpu.VMEM((BLK * HIST,), jnp.int32),
        pltpu.VMEM((2, BLK), jnp.int32),
        pltpu.VMEM((2, BLK, BLK), jnp.float32),
        pltpu.VMEM((2, EMBED_DIM, BLK), jnp.float32),
        pltpu.SemaphoreType.DMA,
        pltpu.SemaphoreType.DMA,
        pltpu.SemaphoreType.DMA,
        pltpu.SemaphoreType.DMA,
    ],
    compiler_params=pltpu.CompilerParams(
        use_tc_tiling_on_sc=True, needs_layout_passes=False
    ),
)
def _sc_gather(idx_hbm, tab_hbm, out_hbm, idx_blk, ipair, prows, otile,
               gsem0, gsem1, wsem0, wsem1):
    wid = lax.axis_index("s") * NUM_CORES + lax.axis_index("c")
    gsems = (gsem0, gsem1)
    wsems = (wsem0, wsem1)
    lane = lax.iota(jnp.int32, 16)

    def build_indices(h, s):
        @pl.loop(0, BLK // 16)
        def _g(g):
            j0 = g * 16
            raw = plsc.load_gather(idx_blk, [(j0 + lane) * HIST + h])
            ipair[s, pl.ds(j0, 16)] = raw

    def start_gather(s):
        pltpu.async_copy(tab_hbm.at[ipair.at[s]], prows.at[s], gsems[s])

    def wait_gather(s):
        pltpu.make_async_copy(
            tab_hbm.at[ipair.at[s]], prows.at[s], gsems[s]
        ).wait()

    perms = [(lane + c) % 16 for c in range(16)]

    def transpose_select(s):
        ot = otile.at[s]
        pr = prows.at[s]

        @pl.loop(0, BLK // 16)
        def _g(g):
            rowv = g * 16 + lane
            for k in range(EMBED_DIM // 16):
                for c in range(16):
                    dvec = k * 16 + perms[c]
                    vals = plsc.load_gather(pr, [rowv, dvec])
                    plsc.store_scatter(ot, [dvec, rowv], vals)

    def start_write(h, b0, s):
        pltpu.async_copy(
            otile.at[s], out_hbm.at[h, :, pl.ds(b0, BLK)], wsems[s]
        )

    def wait_write(h, b0, s):
        pltpu.make_async_copy(
            otile.at[s], out_hbm.at[h, :, pl.ds(b0, BLK)], wsems[s]
        ).wait()

    @pl.loop(0, BLOCKS_PER_WORKER)
    def _blk(bi):
        blk_id = wid * BLOCKS_PER_WORKER + bi
        b0 = blk_id * BLK
        pltpu.sync_copy(idx_hbm.at[pl.ds(b0 * HIST, BLK * HIST)], idx_blk)

        build_indices(0, 0)
        start_gather(0)

        @pl.loop(0, HIST, step=2)
        def _h(h0):
            build_indices(h0 + 1, 1)
            start_gather(1)
            wait_gather(0)

            @pl.when(h0 >= 2)
            def _():
                wait_write(h0 - 2, b0, 0)

            transpose_select(0)
            start_write(h0, b0, 0)

            @pl.when(h0 + 2 < HIST)
            def _():
                build_indices(h0 + 2, 0)
                start_gather(0)

            wait_gather(1)

            @pl.when(h0 >= 1)
            def _():
                wait_write(h0 - 1, b0, 1)

            transpose_select(1)
            start_write(h0 + 1, b0, 1)

        wait_write(HIST - 2, b0, 0)
        wait_write(HIST - 1, b0, 1)


def kernel(indices, table):
    idx = indices.reshape(-1).astype(jnp.int32)
    tabp = jnp.pad(table, ((0, 0), (0, 64)))
    out_t = _sc_gather(idx, tabp)
    return jnp.transpose(out_t, (2, 0, 1))
